# Initial kernel scaffold; baseline (speedup 1.0000x reference)
import functools

import jax
import jax.numpy as jnp
from jax import lax
from jax.experimental import pallas as pl
from jax.experimental.pallas import tpu as pltpu
from jax.experimental.pallas import tpu_sc as plsc

B, S, D, V = 4096, 200, 64, 100000
NC, NS = 2, 16
NW = NC * NS
BPW = B // NW
NB = 4
NCHUNK = BPW // NB
SO = S + 1


def _sc_body(x_hbm, table_hbm, cls_hbm, pos_hbm, out_hbm,
             idx_v, buf, pos_v, cls_v, sem):
    wid = lax.axis_index("s") * NC + lax.axis_index("c")

    pltpu.sync_copy(pos_hbm, pos_v)
    pltpu.sync_copy(cls_hbm, cls_v)

    for k in range(D // 16):
        ck = cls_v[0, pl.ds(16 * k, 16)]
        for bi in range(NB):
            buf[bi * SO, pl.ds(16 * k, 16)] = ck

    def chunk_body(c, _):
        b0 = wid * BPW + c * NB
        pltpu.sync_copy(x_hbm.at[pl.ds(2 * b0, 2 * NB)], idx_v)

        copies = []
        for i in range(2 * NB):
            dst_row = (i // 2) * SO + 1 + (i % 2) * 100
            copies.append(
                pltpu.async_copy(
                    table_hbm.at[idx_v.at[i]],
                    buf.at[pl.ds(dst_row, 100)],
                    sem,
                )
            )
        for cp in copies:
            cp.wait()

        def s_body(s, _):
            for k in range(D // 16):
                p = pos_v[s, pl.ds(16 * k, 16)]
                for bi in range(NB):
                    r = bi * SO + 1 + s
                    buf[r, pl.ds(16 * k, 16)] = buf[r, pl.ds(16 * k, 16)] + p
            return 0

        lax.fori_loop(0, S, s_body, 0)

        pltpu.sync_copy(buf, out_hbm.at[pl.ds(b0 * SO, NB * SO)])
        return 0

    lax.fori_loop(0, NCHUNK, chunk_body, 0)


@jax.jit
def _embed(x2, value_table, cls2, pos_embedding):
    mesh = plsc.VectorSubcoreMesh(core_axis_name="c", subcore_axis_name="s")
    f = pl.kernel(
        _sc_body,
        out_type=jax.ShapeDtypeStruct((B * SO, D), jnp.float32),
        mesh=mesh,
        scratch_types=[
            pltpu.VMEM((2 * NB, S // 2), jnp.int32),
            pltpu.VMEM((NB * SO, D), jnp.float32),
            pltpu.VMEM((S, D), jnp.float32),
            pltpu.VMEM((1, D), jnp.float32),
            pltpu.SemaphoreType.DMA,
        ],
    )
    return f(x2, value_table, cls2, pos_embedding)


def kernel(x, value_table, cls_token, pos_embedding):
    x2 = x.astype(jnp.int32).reshape(2 * B, S // 2)
    cls2 = cls_token.reshape(1, D)
    out = _embed(x2, value_table, cls2, pos_embedding)
    return out.reshape(B, SO, D)


# speedup vs baseline: 3.9919x; 3.9919x over previous
import functools

import jax
import jax.numpy as jnp
from jax import lax
from jax.experimental import pallas as pl
from jax.experimental.pallas import tpu as pltpu
from jax.experimental.pallas import tpu_sc as plsc

B, S, D, V = 4096, 200, 64, 100000
NC, NS = 2, 16
NW = NC * NS
BPW = B // NW
NB = 8
NCHUNK = BPW // NB
SO = S + 1


def _sc_body(x_hbm, table_hbm, cls_hbm, pos_hbm, out_hbm,
             idx_v, buf, pos_v, cls_v, sem):
    wid = lax.axis_index("s") * NC + lax.axis_index("c")

    pltpu.sync_copy(pos_hbm, pos_v)
    pltpu.sync_copy(cls_hbm, cls_v)

    for k in range(D // 16):
        ck = cls_v[0, pl.ds(16 * k, 16)]
        for bi in range(NB):
            buf[bi * SO, pl.ds(16 * k, 16)] = ck

    def chunk_body(c, _):
        b0 = wid * BPW + c * NB
        pltpu.sync_copy(x_hbm.at[pl.ds(2 * b0, 2 * NB)], idx_v)

        copies = []
        for i in range(2 * NB):
            dst_row = (i // 2) * SO + 1 + (i % 2) * 100
            copies.append(
                pltpu.async_copy(
                    table_hbm.at[idx_v.at[i]],
                    buf.at[pl.ds(dst_row, 100)],
                    sem,
                )
            )
        for cp in copies:
            cp.wait()

        def s_body(s, _):
            for k in range(D // 16):
                p = pos_v[s, pl.ds(16 * k, 16)]
                for bi in range(NB):
                    r = bi * SO + 1 + s
                    buf[r, pl.ds(16 * k, 16)] = buf[r, pl.ds(16 * k, 16)] + p
            return 0

        lax.fori_loop(0, S, s_body, 0)

        pltpu.sync_copy(buf, out_hbm.at[pl.ds(b0 * SO, NB * SO)])
        return 0

    lax.fori_loop(0, NCHUNK, chunk_body, 0)


@jax.jit
def _embed(x2, value_table, cls2, pos_embedding):
    mesh = plsc.VectorSubcoreMesh(core_axis_name="c", subcore_axis_name="s")
    f = pl.kernel(
        _sc_body,
        out_type=jax.ShapeDtypeStruct((B * SO, D), jnp.float32),
        mesh=mesh,
        scratch_types=[
            pltpu.VMEM((2 * NB, S // 2), jnp.int32),
            pltpu.VMEM((NB * SO, D), jnp.float32),
            pltpu.VMEM((S, D), jnp.float32),
            pltpu.VMEM((1, D), jnp.float32),
            pltpu.SemaphoreType.DMA,
        ],
        compiler_params=pltpu.CompilerParams(use_tc_tiling_on_sc=False),
    )
    return f(x2, value_table, cls2, pos_embedding)


def kernel(x, value_table, cls_token, pos_embedding):
    x2 = x.astype(jnp.int32).reshape(2 * B, S // 2)
    cls2 = cls_token.reshape(1, D)
    out = _embed(x2, value_table, cls2, pos_embedding)
    return out.reshape(B, SO, D)
